# baseline (speedup 1.0000x reference)
import functools

import jax
import jax.numpy as jnp
from jax import lax
from jax.experimental import pallas as pl
from jax.experimental.pallas import tpu as pltpu
from jax.experimental.pallas import tpu_sc as plsc

L = 16
NC = 2
NS = 16
NW = NC * NS
E = 64
BLK = 4096
SW = 128
SC_BLKS = 68
NBUF = 8
SCN = NW * SC_BLKS * SW


def _matvec2_body(w_ref, tu_ref, tp_ref, su_ref, sp_ref):
  w = w_ref[...]
  wu = w[0, :E].reshape(E, 1)
  wp = w[0, E:].reshape(E, 1)
  su_ref[...] = jnp.sum(tu_ref[...] * wu, axis=0)
  sp_ref[...] = jnp.sum(tp_ref[...] * wp, axis=0)


@functools.lru_cache(maxsize=None)
def _matvec2(nblocks, blk_off, out_n):
  return pl.pallas_call(
      _matvec2_body,
      grid=(nblocks,),
      in_specs=[
          pl.BlockSpec((1, 2 * E), lambda i: (0, 0)),
          pl.BlockSpec((E, BLK), lambda i: (0, blk_off + i)),
          pl.BlockSpec((E, BLK), lambda i: (0, blk_off + i)),
      ],
      out_specs=[
          pl.BlockSpec((BLK,), lambda i: (i,)),
          pl.BlockSpec((BLK,), lambda i: (i,)),
      ],
      out_shape=[
          jax.ShapeDtypeStruct((out_n,), jnp.float32),
          jax.ShapeDtypeStruct((out_n,), jnp.float32),
      ],
  )


@functools.lru_cache(maxsize=None)
def _sc_dense():
  per_w = SC_BLKS * SW
  steps = 2 * SC_BLKS
  mesh = plsc.VectorSubcoreMesh(core_axis_name="c", subcore_axis_name="s")

  @functools.partial(
      pl.kernel,
      out_type=[jax.ShapeDtypeStruct((SCN,), jnp.float32),
                jax.ShapeDtypeStruct((SCN,), jnp.float32)],
      mesh=mesh,
      compiler_params=pltpu.CompilerParams(use_tc_tiling_on_sc=True),
      scratch_types=(
          [pltpu.VMEM((E, SW), jnp.float32)] * NBUF
          + [pltpu.VMEM((2 * E,), jnp.float32),
             pltpu.VMEM((per_w,), jnp.float32),
             pltpu.VMEM((per_w,), jnp.float32)]
          + [pltpu.SemaphoreType.DMA] * NBUF
      ),
  )
  def k(tu_hbm, tp_hbm, w_hbm, su_hbm, sp_hbm, *rest):
    bufs = rest[:NBUF]
    w_v, ou_v, op_v = rest[NBUF:NBUF + 3]
    sems = rest[NBUF + 3:]
    wid = lax.axis_index("s") * NC + lax.axis_index("c")
    base = wid * per_w
    pltpu.sync_copy(w_hbm.at[0], w_v)

    wv = [w_v[pl.ds(16 * q, L)] for q in range(2 * E // L)]
    ws = [wv[c // L][c % L] for c in range(2 * E)]
    wgt = (ws[:E], ws[E:])

    tbls = (tu_hbm, tp_hbm)
    outs = (ou_v, op_v)

    def start(par, g, buf, sem):
      off = pl.multiple_of(base + g * SW, SW)
      pltpu.async_copy(tbls[par].at[:, pl.ds(off, SW)], buf, sem)

    for b in range(NBUF):
      start(b % 2, b // 2, bufs[b], sems[b])

    def compute(g, par, buf):
      out_v, w_ = outs[par], wgt[par]
      for grp in range(SW // L):
        col = pl.ds(grp * L, L)
        accs = [buf[q, col] * w_[q] for q in range(4)]
        for c in range(4, E):
          accs[c % 4] += buf[c, col] * w_[c]
        out_v[pl.ds(g * SW + grp * L, L)] = (
            (accs[0] + accs[1]) + (accs[2] + accs[3]))

    @pl.loop(0, steps, step=NBUF)
    def _g(j0):
      for b in range(NBUF):
        g = j0 // 2 + b // 2
        off = pl.multiple_of(base + g * SW, SW)
        pltpu.make_async_copy(tbls[b % 2].at[:, pl.ds(off, SW)],
                              bufs[b], sems[b]).wait()
        compute(g, b % 2, bufs[b])
        @pl.when(j0 + b + NBUF < steps)
        def _():
          start(b % 2, g + NBUF // 2, bufs[b], sems[b])

    pltpu.sync_copy(ou_v, su_hbm.at[pl.ds(base, per_w)])
    pltpu.sync_copy(op_v, sp_hbm.at[pl.ds(base, per_w)])

  return k


@functools.lru_cache(maxsize=None)
def _gather(B):
  BW = B // NW
  NCH = BW // 128
  mesh = plsc.VectorSubcoreMesh(core_axis_name="c", subcore_axis_name="s")

  @functools.partial(
      pl.kernel,
      out_type=jax.ShapeDtypeStruct((B,), jnp.float32),
      mesh=mesh,
      compiler_params=pltpu.CompilerParams(use_tc_tiling_on_sc=False),
      scratch_types=[
          pltpu.VMEM((BW,), jnp.int32),
          pltpu.VMEM((BW,), jnp.int32),
          pltpu.VMEM((NCH, 128), jnp.int32),
          pltpu.VMEM((NCH, 128), jnp.int32),
          pltpu.VMEM((BW, L), jnp.float32),
          pltpu.VMEM((BW, L), jnp.float32),
          pltpu.VMEM((L,), jnp.float32),
          pltpu.VMEM((BW,), jnp.float32),
          pltpu.SemaphoreType.DMA,
          pltpu.SemaphoreType.DMA,
      ],
  )
  def k(user_hbm, power_hbm, su_hbm, sp_hbm, fcb_hbm, out_hbm,
        uidx_v, pidx_v, uq_v, pq_v, ubuf_v, pbuf_v, b_v, out_v, usem, psem):
    wid = lax.axis_index("s") * NC + lax.axis_index("c")
    base = wid * BW

    pltpu.sync_copy(user_hbm.at[pl.ds(base, BW)], uidx_v)
    pltpu.sync_copy(power_hbm.at[pl.ds(base, BW)], pidx_v)
    pltpu.sync_copy(fcb_hbm, b_v.at[pl.ds(0, 1)])

    @plsc.parallel_loop(0, BW // L, 1, unroll=4)
    def _mkq(g):
      off = g * L
      uq_v[off // 128, pl.ds(off % 128, L)] = (
          lax.shift_right_logical(uidx_v[pl.ds(off, L)], 4))
      pq_v[off // 128, pl.ds(off % 128, L)] = (
          lax.shift_right_logical(pidx_v[pl.ds(off, L)], 4))

    for j in range(NCH):
      pltpu.async_copy(su_hbm.at[uq_v.at[j]],
                       ubuf_v.at[pl.ds(j * 128, 128)], usem)
      pltpu.async_copy(sp_hbm.at[pq_v.at[j]],
                       pbuf_v.at[pl.ds(j * 128, 128)], psem)
    for j in range(NCH):
      pltpu.make_async_copy(su_hbm.at[uq_v.at[j]],
                            ubuf_v.at[pl.ds(j * 128, 128)], usem).wait()
      pltpu.make_async_copy(sp_hbm.at[pq_v.at[j]],
                            pbuf_v.at[pl.ds(j * 128, 128)], psem).wait()

    lanes = lax.iota(jnp.int32, L)
    dnums = lax.GatherDimensionNumbers(
        offset_dims=(), collapsed_slice_dims=(0,), start_index_map=(0,))

    def _pick(v, m):
      idx = jnp.broadcast_to(m, (L,)).astype(jnp.int32)
      return lax.gather(v, idx[:, None], dnums, (1,),
                        mode=lax.GatherScatterMode.PROMISE_IN_BOUNDS)

    b = b_v[...][0]

    @plsc.parallel_loop(0, BW // L, 1, unroll=2)
    def _grp(g):
      off = g * L
      um = uidx_v[pl.ds(off, L)] & (L - 1)
      pm = pidx_v[pl.ds(off, L)] & (L - 1)
      out = jnp.zeros((L,), jnp.float32)
      for j in range(L):
        uv = ubuf_v[off + j, :]
        pv = pbuf_v[off + j, :]
        s = _pick(uv, um[j]) + _pick(pv, pm[j])
        out = jnp.where(lanes == j, s, out)
      out_v[pl.ds(off, L)] = out + b

    pltpu.sync_copy(out_v, out_hbm.at[pl.ds(base, BW)])

  return k


def kernel(user, power, user_table, power_table, fc_w, fc_b):
  n = user_table.shape[0]
  hi_blk = SCN // BLK
  hi_n = n - SCN
  hi_grid = (hi_n + BLK - 1) // BLK

  su_lo, sp_lo = _sc_dense()(user_table.T, power_table.T, fc_w)
  su_hi, sp_hi = _matvec2(hi_grid, hi_blk, hi_n)(
      fc_w, user_table.T, power_table.T)
  su = jnp.concatenate([su_lo, su_hi])
  sp = jnp.concatenate([sp_lo, sp_hi])

  nl = n // L
  out = _gather(user.shape[0])(user, power, su.reshape(nl, L),
                               sp.reshape(nl, L), fc_b)
  return out


# speedup vs baseline: 1.1598x; 1.1598x over previous
import functools

import jax
import jax.numpy as jnp
from jax import lax
from jax.experimental import pallas as pl
from jax.experimental.pallas import tpu as pltpu
from jax.experimental.pallas import tpu_sc as plsc

L = 16
NC = 2
NS = 16
NW = NC * NS
E = 64
BLK = 4096
SW = 128
SC_BLKS = 68
NBUF = 4
SCN = NW * SC_BLKS * SW


def _matvec2_body(w_ref, tu_ref, tp_ref, su_ref, sp_ref):
  w = w_ref[...]
  wu = w[0, :E].reshape(E, 1)
  wp = w[0, E:].reshape(E, 1)
  su_ref[...] = jnp.sum(tu_ref[...] * wu, axis=0)
  sp_ref[...] = jnp.sum(tp_ref[...] * wp, axis=0)


@functools.lru_cache(maxsize=None)
def _matvec2(nblocks, blk_off, out_n):
  return pl.pallas_call(
      _matvec2_body,
      grid=(nblocks,),
      in_specs=[
          pl.BlockSpec((1, 2 * E), lambda i: (0, 0)),
          pl.BlockSpec((E, BLK), lambda i: (0, blk_off + i)),
          pl.BlockSpec((E, BLK), lambda i: (0, blk_off + i)),
      ],
      out_specs=[
          pl.BlockSpec((BLK,), lambda i: (i,)),
          pl.BlockSpec((BLK,), lambda i: (i,)),
      ],
      out_shape=[
          jax.ShapeDtypeStruct((out_n,), jnp.float32),
          jax.ShapeDtypeStruct((out_n,), jnp.float32),
      ],
  )


@functools.lru_cache(maxsize=None)
def _sc_dense():
  per_w = SC_BLKS * SW
  steps = 2 * SC_BLKS
  mesh = plsc.VectorSubcoreMesh(core_axis_name="c", subcore_axis_name="s")

  @functools.partial(
      pl.kernel,
      out_type=[jax.ShapeDtypeStruct((SCN,), jnp.float32),
                jax.ShapeDtypeStruct((SCN,), jnp.float32)],
      mesh=mesh,
      compiler_params=pltpu.CompilerParams(use_tc_tiling_on_sc=True),
      scratch_types=(
          [pltpu.VMEM((E, SW), jnp.float32)] * NBUF
          + [pltpu.VMEM((2 * E,), jnp.float32),
             pltpu.VMEM((per_w,), jnp.float32),
             pltpu.VMEM((per_w,), jnp.float32)]
          + [pltpu.SemaphoreType.DMA] * NBUF
      ),
  )
  def k(tu_hbm, tp_hbm, w_hbm, su_hbm, sp_hbm, *rest):
    bufs = rest[:NBUF]
    w_v, ou_v, op_v = rest[NBUF:NBUF + 3]
    sems = rest[NBUF + 3:]
    wid = lax.axis_index("s") * NC + lax.axis_index("c")
    base = wid * per_w
    pltpu.sync_copy(w_hbm.at[0], w_v)

    wv = [w_v[pl.ds(16 * q, L)] for q in range(2 * E // L)]
    ws = [wv[c // L][c % L] for c in range(2 * E)]
    wgt = (ws[:E], ws[E:])

    tbls = (tu_hbm, tp_hbm)
    outs = (ou_v, op_v)

    def start(par, g, buf, sem):
      off = pl.multiple_of(base + g * SW, SW)
      pltpu.async_copy(tbls[par].at[:, pl.ds(off, SW)], buf, sem)

    for b in range(NBUF):
      start(b % 2, b // 2, bufs[b], sems[b])

    def compute(g, par, buf):
      out_v, w_ = outs[par], wgt[par]
      for grp in range(SW // L):
        col = pl.ds(grp * L, L)
        accs = [buf[q, col] * w_[q] for q in range(4)]
        for c in range(4, E):
          accs[c % 4] += buf[c, col] * w_[c]
        out_v[pl.ds(g * SW + grp * L, L)] = (
            (accs[0] + accs[1]) + (accs[2] + accs[3]))

    @pl.loop(0, steps, step=NBUF)
    def _g(j0):
      for b in range(NBUF):
        g = j0 // 2 + b // 2
        off = pl.multiple_of(base + g * SW, SW)
        pltpu.make_async_copy(tbls[b % 2].at[:, pl.ds(off, SW)],
                              bufs[b], sems[b]).wait()
        compute(g, b % 2, bufs[b])
        @pl.when(j0 + b + NBUF < steps)
        def _():
          start(b % 2, g + NBUF // 2, bufs[b], sems[b])

    pltpu.sync_copy(ou_v, su_hbm.at[pl.ds(base, per_w)])
    pltpu.sync_copy(op_v, sp_hbm.at[pl.ds(base, per_w)])

  return k


@functools.lru_cache(maxsize=None)
def _gather(B):
  BW = B // NW
  NCH = BW // 128
  mesh = plsc.VectorSubcoreMesh(core_axis_name="c", subcore_axis_name="s")

  @functools.partial(
      pl.kernel,
      out_type=jax.ShapeDtypeStruct((B,), jnp.float32),
      mesh=mesh,
      compiler_params=pltpu.CompilerParams(use_tc_tiling_on_sc=False),
      scratch_types=[
          pltpu.VMEM((BW,), jnp.int32),
          pltpu.VMEM((BW,), jnp.int32),
          pltpu.VMEM((NCH, 128), jnp.int32),
          pltpu.VMEM((NCH, 128), jnp.int32),
          pltpu.VMEM((BW, L), jnp.float32),
          pltpu.VMEM((BW, L), jnp.float32),
          pltpu.VMEM((L,), jnp.float32),
          pltpu.VMEM((BW,), jnp.float32),
          pltpu.SemaphoreType.DMA,
          pltpu.SemaphoreType.DMA,
      ],
  )
  def k(user_hbm, power_hbm, su_hbm, sp_hbm, fcb_hbm, out_hbm,
        uidx_v, pidx_v, uq_v, pq_v, ubuf_v, pbuf_v, b_v, out_v, usem, psem):
    wid = lax.axis_index("s") * NC + lax.axis_index("c")
    base = wid * BW

    pltpu.sync_copy(user_hbm.at[pl.ds(base, BW)], uidx_v)
    pltpu.sync_copy(power_hbm.at[pl.ds(base, BW)], pidx_v)
    pltpu.sync_copy(fcb_hbm, b_v.at[pl.ds(0, 1)])

    @plsc.parallel_loop(0, BW // L, 1, unroll=4)
    def _mkq(g):
      off = g * L
      uq_v[off // 128, pl.ds(off % 128, L)] = (
          lax.shift_right_logical(uidx_v[pl.ds(off, L)], 4))
      pq_v[off // 128, pl.ds(off % 128, L)] = (
          lax.shift_right_logical(pidx_v[pl.ds(off, L)], 4))

    for j in range(NCH):
      pltpu.async_copy(su_hbm.at[uq_v.at[j]],
                       ubuf_v.at[pl.ds(j * 128, 128)], usem)
      pltpu.async_copy(sp_hbm.at[pq_v.at[j]],
                       pbuf_v.at[pl.ds(j * 128, 128)], psem)
    for j in range(NCH):
      pltpu.make_async_copy(su_hbm.at[uq_v.at[j]],
                            ubuf_v.at[pl.ds(j * 128, 128)], usem).wait()
      pltpu.make_async_copy(sp_hbm.at[pq_v.at[j]],
                            pbuf_v.at[pl.ds(j * 128, 128)], psem).wait()

    lanes = lax.iota(jnp.int32, L)
    dnums = lax.GatherDimensionNumbers(
        offset_dims=(), collapsed_slice_dims=(0,), start_index_map=(0,))

    def _pick(v, m):
      idx = jnp.broadcast_to(m, (L,)).astype(jnp.int32)
      return lax.gather(v, idx[:, None], dnums, (1,),
                        mode=lax.GatherScatterMode.PROMISE_IN_BOUNDS)

    b = b_v[...][0]

    @plsc.parallel_loop(0, BW // L, 1, unroll=2)
    def _grp(g):
      off = g * L
      um = uidx_v[pl.ds(off, L)] & (L - 1)
      pm = pidx_v[pl.ds(off, L)] & (L - 1)
      out = jnp.zeros((L,), jnp.float32)
      for j in range(L):
        uv = ubuf_v[off + j, :]
        pv = pbuf_v[off + j, :]
        s = _pick(uv, um[j]) + _pick(pv, pm[j])
        out = jnp.where(lanes == j, s, out)
      out_v[pl.ds(off, L)] = out + b

    pltpu.sync_copy(out_v, out_hbm.at[pl.ds(base, BW)])

  return k


def kernel(user, power, user_table, power_table, fc_w, fc_b):
  n = user_table.shape[0]
  hi_blk = SCN // BLK
  hi_n = n - SCN
  hi_grid = (hi_n + BLK - 1) // BLK

  su_lo, sp_lo = _sc_dense()(user_table.T, power_table.T, fc_w)
  su_hi, sp_hi = _matvec2(hi_grid, hi_blk, hi_n)(
      fc_w, user_table.T, power_table.T)
  su = jnp.concatenate([su_lo, su_hi])
  sp = jnp.concatenate([sp_lo, sp_hi])

  nl = n // L
  out = _gather(user.shape[0])(user, power, su.reshape(nl, L),
                               sp.reshape(nl, L), fc_b)
  return out
